# baseline (speedup 1.0000x reference)
import jax
import jax.numpy as jnp
from jax.experimental import pallas as pl
from jax.experimental.pallas import tpu as pltpu

N = 8192
G = 256
NPG = 32
BLK = 4096
GPB = BLK // NPG
NB = N // BLK
EPS = 1e-5
H1 = 256
H2 = 256
P = 2 * (H1 + H2)
L0 = 512
L1 = 256
L2 = 128
NCAT = 32
D = 128
QB = 256


def _dot(a, b):
    return jnp.dot(a, b, preferred_element_type=jnp.float32)


def _fused(x_ref, adj0_ref, adj1_ref, adj2_ref, adj3_ref,
           adj4_ref, adj5_ref, adj6_ref, adj7_ref,
           adj8_ref, adj9_ref, adj10_ref, adj11_ref,
           adj12_ref, adj13_ref, adj14_ref, adj15_ref,
           w1_ref, ws1_ref, w2_ref, ws2_ref,
           l1w_ref, l2w_ref, l3w_ref, cw_ref,
           b1_ref, g1_ref, bb1_ref, b2_ref, g2_ref, bb2_ref,
           g0_ref, b0_ref, l1b_ref, hg1_ref, hb1_ref,
           l2b_ref, hg2_ref, hb2_ref, l3b_ref, cb_ref,
           out_ref, outc_ref, fp_ref,
           r1_s, adj_s, pa_s, pb_s, st1_s, st2_s, wc1_s, wc2_s, ss_s):
    i = pl.program_id(0)

    def bn_affine(st_ref, g, b):
        m = st_ref[0:1, :] / N
        v = st_ref[1:2, :] / N - m * m
        scale = g * jax.lax.rsqrt(v + EPS)
        return scale, b - m * scale

    @pl.when(i < NB)
    def _phase_a():
        blk = i

        @pl.when(i == 0)
        def _():
            st1_s[...] = jnp.zeros_like(st1_s)
            wc1_s[:, 0:H1] = w1_ref[...]
            wc1_s[:, H1:2 * H1] = ws1_ref[...]
            wc2_s[:, 0:H2] = w2_ref[...]
            wc2_s[:, H2:2 * H2] = ws2_ref[...]
            io0 = jax.lax.broadcasted_iota(jnp.int32, (16, QB), 0)
            io1 = jax.lax.broadcasted_iota(jnp.int32, (16, QB), 1)
            ss_s[...] = jnp.where((io1 // NPG == io0) | (io0 == 8), 1.0, 0.0)

        ps = jnp.zeros((1, H1), jnp.float32)
        pss = jnp.zeros((1, H1), jnp.float32)
        for q, aq in enumerate((adj0_ref, adj1_ref, adj2_ref, adj3_ref,
                                adj4_ref, adj5_ref, adj6_ref, adj7_ref,
                                adj8_ref, adj9_ref, adj10_ref, adj11_ref,
                                adj12_ref, adj13_ref, adj14_ref, adj15_ref)):
            xq = x_ref[q * QB:(q + 1) * QB, :]
            uvq = _dot(xq, wc1_s[...])
            ab = aq[...]
            adj_s[pl.ds(blk * BLK + q * QB, QB), :] = ab
            o = _dot(ab, uvq[:, 0:H1]) + b1_ref[...] + uvq[:, H1:2 * H1]
            r = jnp.maximum(o, 0.0)
            r1_s[pl.ds(blk * BLK + q * QB, QB), :] = r
            ps += _dot(ss_s[8:9, :], r)
            pss += _dot(ss_s[8:9, :], r * r)

        st1_s[0:1, :] += ps
        st1_s[1:2, :] += pss

    @pl.when((i >= NB) & (i < 2 * NB))
    def _phase_b():
        blk = i - NB
        scale, shift = bn_affine(st1_s, g1_ref[...], bb1_ref[...])

        @pl.when(i == NB)
        def _():
            st2_s[...] = jnp.zeros_like(st2_s)

        ps = jnp.zeros((1, H2), jnp.float32)
        pss = jnp.zeros((1, H2), jnp.float32)
        for q in range(BLK // QB):
            hq = r1_s[pl.ds(blk * BLK + q * QB, QB), :] * scale + shift
            uvq = _dot(hq, wc2_s[...])
            ab = adj_s[pl.ds(blk * BLK + q * QB, QB), :]
            o = _dot(ab, uvq[:, 0:H2]) + b2_ref[...] + uvq[:, H2:2 * H2]
            r = jnp.maximum(o, 0.0)
            sums_h = _dot(ss_s[...], hq)
            sums_r = _dot(ss_s[...], r)
            ps += sums_r[8:9, :]
            pss += _dot(ss_s[8:9, :], r * r)
            grow0 = blk * GPB + q * (QB // NPG)
            pa_s[pl.ds(grow0, 8), H1:2 * H1] = sums_h[0:8, :]
            pb_s[pl.ds(grow0, 8), 2 * H2:3 * H2] = sums_r[0:8, :]
            for g in range(QB // NPG):
                hg = hq[g * NPG:(g + 1) * NPG, :]
                rg = r[g * NPG:(g + 1) * NPG, :]
                grow = grow0 + g
                pa_s[pl.ds(grow, 1), 0:H1] = jnp.max(hg, axis=0, keepdims=True)
                pb_s[pl.ds(grow, 1), 0:H2] = jnp.max(rg, axis=0, keepdims=True)
                pb_s[pl.ds(grow, 1), H2:2 * H2] = jnp.min(rg, axis=0, keepdims=True)

        st2_s[0:1, :] += ps
        st2_s[1:2, :] += pss

    @pl.when(i == pl.num_programs(0) - 1)
    def _phase_d():
        def bn(t, g, b):
            m = jnp.mean(t, axis=0, keepdims=True)
            v = jnp.mean(t * t, axis=0, keepdims=True) - m * m
            return (t - m) * (g * jax.lax.rsqrt(v + EPS)) + b

        e1max = pa_s[:, 0:H1]
        e1mean = pa_s[:, H1:2 * H1] * (1.0 / NPG)
        sc2, sh2 = bn_affine(st2_s, g2_ref[...], bb2_ref[...])
        h2max = jnp.where(sc2 > 0, sc2 * pb_s[:, 0:H2],
                          sc2 * pb_s[:, H2:2 * H2]) + sh2
        h2mean = sc2 * (pb_s[:, 2 * H2:3 * H2] * (1.0 / NPG)) + sh2
        pooled = jnp.concatenate([e1max, h2max, e1mean, h2mean], axis=1)
        p = bn(pooled, g0_ref[...], b0_ref[...])
        p = jnp.maximum(_dot(p, l1w_ref[...]) + l1b_ref[...], 0.0)
        p = bn(p, hg1_ref[...], hb1_ref[...])
        p = jnp.maximum(_dot(p, l2w_ref[...]) + l2b_ref[...], 0.0)
        fp = bn(p, hg2_ref[...], hb2_ref[...])
        fp_ref[...] = fp
        out_ref[...] = _dot(fp, l3w_ref[...]) + l3b_ref[...]
        outc_ref[...] = _dot(fp, cw_ref[...]) + cb_ref[...]


def kernel(x, adj, slice_list, W1, Wskip1, b1, W2, Wskip2, b2, bng1_g, bng1_b,
           bng2_g, bng2_b, bn0_g, bn0_b, lin1_W, lin1_b, bn1_g, bn1_b, lin2_W,
           lin2_b, bn2_g, bn2_b, lin3_W, lin3_b, cat_W, cat_b):
    row = lambda a: a.reshape(1, -1)
    full = lambda a: pl.BlockSpec(a.shape, lambda i: (0,) * a.ndim)

    def x_map(i):
        j = jnp.minimum(i, NB - 1)
        return (j, 0)

    def adj_map(q):
        def m(i):
            j = jnp.minimum(i, NB - 1) * (BLK // QB) + q
            return (j, j)
        return m

    args = (x,) + (adj,) * 16 + (W1, Wskip1, W2, Wskip2,
            lin1_W, lin2_W, lin3_W, cat_W,
            row(b1), row(bng1_g), row(bng1_b), row(b2), row(bng2_g),
            row(bng2_b), row(bn0_g), row(bn0_b), row(lin1_b), row(bn1_g),
            row(bn1_b), row(lin2_b), row(bn2_g), row(bn2_b), row(lin3_b),
            row(cat_b))
    in_specs = [
        pl.BlockSpec((BLK, D), x_map),
    ] + [pl.BlockSpec((QB, QB), adj_map(q)) for q in range(BLK // QB)
    ] + [full(a) for a in args[1 + BLK // QB:]]

    out, out_class, fp = pl.pallas_call(
        _fused,
        grid=(2 * NB + 1,),
        in_specs=in_specs,
        out_specs=[
            pl.BlockSpec((G, L2), lambda i: (0, 0)),
            pl.BlockSpec((G, NCAT), lambda i: (0, 0)),
            pl.BlockSpec((G, L1), lambda i: (0, 0)),
        ],
        out_shape=[
            jax.ShapeDtypeStruct((G, L2), jnp.float32),
            jax.ShapeDtypeStruct((G, NCAT), jnp.float32),
            jax.ShapeDtypeStruct((G, L1), jnp.float32),
        ],
        scratch_shapes=[
            pltpu.VMEM((N, H1), jnp.float32),
            pltpu.VMEM((N, QB), jnp.float32),
            pltpu.VMEM((G, 2 * H1), jnp.float32),
            pltpu.VMEM((G, 3 * H2), jnp.float32),
            pltpu.VMEM((8, H1), jnp.float32),
            pltpu.VMEM((8, H2), jnp.float32),
            pltpu.VMEM((D, 2 * H1), jnp.float32),
            pltpu.VMEM((H1, 2 * H2), jnp.float32),
            pltpu.VMEM((16, QB), jnp.float32),
        ],
        compiler_params=pltpu.CompilerParams(
            dimension_semantics=("arbitrary",),
        ),
    )(*args)

    return (out, out_class, fp)


# speedup vs baseline: 1.2477x; 1.2477x over previous
import jax
import jax.numpy as jnp
from jax.experimental import pallas as pl
from jax.experimental.pallas import tpu as pltpu

N = 8192
G = 256
NPG = 32
BLK = 4096
GPB = BLK // NPG
NB = N // BLK
EPS = 1e-5
H1 = 256
H2 = 256
P = 2 * (H1 + H2)
L0 = 512
L1 = 256
L2 = 128
NCAT = 32
D = 128
QB = 256


def _dot(a, b):
    return jnp.dot(a, b, preferred_element_type=jnp.float32)


def _fused(x_ref, adj0_ref, adj1_ref, adj2_ref, adj3_ref,
           adj4_ref, adj5_ref, adj6_ref, adj7_ref,
           adj8_ref, adj9_ref, adj10_ref, adj11_ref,
           adj12_ref, adj13_ref, adj14_ref, adj15_ref,
           w1_ref, ws1_ref, w2_ref, ws2_ref,
           l1w_ref, l2w_ref, l3w_ref, cw_ref,
           b1_ref, g1_ref, bb1_ref, b2_ref, g2_ref, bb2_ref,
           g0_ref, b0_ref, l1b_ref, hg1_ref, hb1_ref,
           l2b_ref, hg2_ref, hb2_ref, l3b_ref, cb_ref,
           out_ref, outc_ref, fp_ref,
           r1_s, adj_s, pa_s, pb_s, st1_s, st2_s, wc1_s, wc2_s):
    i = pl.program_id(0)

    def bn_affine(st_ref, g, b):
        m = st_ref[0:1, :] / N
        v = st_ref[1:2, :] / N - m * m
        scale = g * jax.lax.rsqrt(v + EPS)
        return scale, b - m * scale

    @pl.when(i < NB)
    def _phase_a():
        blk = i

        @pl.when(i == 0)
        def _():
            st1_s[...] = jnp.zeros_like(st1_s)
            wc1_s[:, 0:H1] = w1_ref[...]
            wc1_s[:, H1:2 * H1] = ws1_ref[...]
            wc2_s[:, 0:H2] = w2_ref[...]
            wc2_s[:, H2:2 * H2] = ws2_ref[...]

        xb = x_ref[...]
        uv = _dot(xb, wc1_s[...])
        ps = jnp.zeros((1, H1), jnp.float32)
        pss = jnp.zeros((1, H1), jnp.float32)
        for q, aq in enumerate((adj0_ref, adj1_ref, adj2_ref, adj3_ref,
                                adj4_ref, adj5_ref, adj6_ref, adj7_ref,
                                adj8_ref, adj9_ref, adj10_ref, adj11_ref,
                                adj12_ref, adj13_ref, adj14_ref, adj15_ref)):
            ab = aq[...]
            adj_s[pl.ds(blk * BLK + q * QB, QB), :] = ab
            o = (_dot(ab, uv[q * QB:(q + 1) * QB, 0:H1]) + b1_ref[...]
                 + uv[q * QB:(q + 1) * QB, H1:2 * H1])
            r = jnp.maximum(o, 0.0)
            r1_s[pl.ds(blk * BLK + q * QB, QB), :] = r
            ps += jnp.sum(r, axis=0, keepdims=True)
            pss += jnp.sum(r * r, axis=0, keepdims=True)

        st1_s[0:1, :] += ps
        st1_s[1:2, :] += pss

    @pl.when((i >= NB) & (i < 2 * NB))
    def _phase_b():
        blk = i - NB
        scale, shift = bn_affine(st1_s, g1_ref[...], bb1_ref[...])
        h = r1_s[pl.ds(blk * BLK, BLK), :] * scale + shift
        uv = _dot(h, wc2_s[...])

        @pl.when(i == NB)
        def _():
            st2_s[...] = jnp.zeros_like(st2_s)

        ps = jnp.zeros((1, H2), jnp.float32)
        pss = jnp.zeros((1, H2), jnp.float32)
        for q in range(BLK // QB):
            ab = adj_s[pl.ds(blk * BLK + q * QB, QB), :]
            o = (_dot(ab, uv[q * QB:(q + 1) * QB, 0:H2]) + b2_ref[...]
                 + uv[q * QB:(q + 1) * QB, H2:2 * H2])
            r = jnp.maximum(o, 0.0)
            ps += jnp.sum(r, axis=0, keepdims=True)
            pss += jnp.sum(r * r, axis=0, keepdims=True)
            for g in range(QB // NPG):
                hg = h[(q * QB + g * NPG):(q * QB + (g + 1) * NPG), :]
                rg = r[g * NPG:(g + 1) * NPG, :]
                grow = blk * GPB + q * (QB // NPG) + g
                pa_s[pl.ds(grow, 1), 0:H1] = jnp.max(hg, axis=0, keepdims=True)
                pa_s[pl.ds(grow, 1), H1:2 * H1] = jnp.sum(hg, axis=0, keepdims=True)
                pb_s[pl.ds(grow, 1), 0:H2] = jnp.max(rg, axis=0, keepdims=True)
                pb_s[pl.ds(grow, 1), H2:2 * H2] = jnp.min(rg, axis=0, keepdims=True)
                pb_s[pl.ds(grow, 1), 2 * H2:3 * H2] = jnp.sum(rg, axis=0, keepdims=True)

        st2_s[0:1, :] += ps
        st2_s[1:2, :] += pss

    @pl.when(i == pl.num_programs(0) - 1)
    def _phase_d():
        def bn(t, g, b):
            m = jnp.mean(t, axis=0, keepdims=True)
            v = jnp.mean(t * t, axis=0, keepdims=True) - m * m
            return (t - m) * (g * jax.lax.rsqrt(v + EPS)) + b

        e1max = pa_s[:, 0:H1]
        e1mean = pa_s[:, H1:2 * H1] * (1.0 / NPG)
        sc2, sh2 = bn_affine(st2_s, g2_ref[...], bb2_ref[...])
        h2max = jnp.where(sc2 > 0, sc2 * pb_s[:, 0:H2],
                          sc2 * pb_s[:, H2:2 * H2]) + sh2
        h2mean = sc2 * (pb_s[:, 2 * H2:3 * H2] * (1.0 / NPG)) + sh2
        pooled = jnp.concatenate([e1max, h2max, e1mean, h2mean], axis=1)
        p = bn(pooled, g0_ref[...], b0_ref[...])
        p = jnp.maximum(_dot(p, l1w_ref[...]) + l1b_ref[...], 0.0)
        p = bn(p, hg1_ref[...], hb1_ref[...])
        p = jnp.maximum(_dot(p, l2w_ref[...]) + l2b_ref[...], 0.0)
        fp = bn(p, hg2_ref[...], hb2_ref[...])
        fp_ref[...] = fp
        out_ref[...] = _dot(fp, l3w_ref[...]) + l3b_ref[...]
        outc_ref[...] = _dot(fp, cw_ref[...]) + cb_ref[...]


def kernel(x, adj, slice_list, W1, Wskip1, b1, W2, Wskip2, b2, bng1_g, bng1_b,
           bng2_g, bng2_b, bn0_g, bn0_b, lin1_W, lin1_b, bn1_g, bn1_b, lin2_W,
           lin2_b, bn2_g, bn2_b, lin3_W, lin3_b, cat_W, cat_b):
    row = lambda a: a.reshape(1, -1)
    full = lambda a: pl.BlockSpec(a.shape, lambda i: (0,) * a.ndim)

    def x_map(i):
        j = jnp.minimum(i, NB - 1)
        return (j, 0)

    def adj_map(q):
        def m(i):
            j = jnp.minimum(i, NB - 1) * (BLK // QB) + q
            return (j, j)
        return m

    args = (x,) + (adj,) * 16 + (W1, Wskip1, W2, Wskip2,
            lin1_W, lin2_W, lin3_W, cat_W,
            row(b1), row(bng1_g), row(bng1_b), row(b2), row(bng2_g),
            row(bng2_b), row(bn0_g), row(bn0_b), row(lin1_b), row(bn1_g),
            row(bn1_b), row(lin2_b), row(bn2_g), row(bn2_b), row(lin3_b),
            row(cat_b))
    in_specs = [
        pl.BlockSpec((BLK, D), x_map),
    ] + [pl.BlockSpec((QB, QB), adj_map(q)) for q in range(BLK // QB)
    ] + [full(a) for a in args[1 + BLK // QB:]]

    out, out_class, fp = pl.pallas_call(
        _fused,
        grid=(2 * NB + 1,),
        in_specs=in_specs,
        out_specs=[
            pl.BlockSpec((G, L2), lambda i: (0, 0)),
            pl.BlockSpec((G, NCAT), lambda i: (0, 0)),
            pl.BlockSpec((G, L1), lambda i: (0, 0)),
        ],
        out_shape=[
            jax.ShapeDtypeStruct((G, L2), jnp.float32),
            jax.ShapeDtypeStruct((G, NCAT), jnp.float32),
            jax.ShapeDtypeStruct((G, L1), jnp.float32),
        ],
        scratch_shapes=[
            pltpu.VMEM((N, H1), jnp.float32),
            pltpu.VMEM((N, QB), jnp.float32),
            pltpu.VMEM((G, 2 * H1), jnp.float32),
            pltpu.VMEM((G, 3 * H2), jnp.float32),
            pltpu.VMEM((8, H1), jnp.float32),
            pltpu.VMEM((8, H2), jnp.float32),
            pltpu.VMEM((D, 2 * H1), jnp.float32),
            pltpu.VMEM((H1, 2 * H2), jnp.float32),
        ],
        compiler_params=pltpu.CompilerParams(
            dimension_semantics=("arbitrary",),
        ),
    )(*args)

    return (out, out_class, fp)
